# baseline (speedup 1.0000x reference)
import functools

import numpy as np
import jax
import jax.numpy as jnp
from jax import lax
from jax.experimental import pallas as pl
from jax.experimental.pallas import tpu as pltpu
from jax.experimental.pallas import tpu_sc as plsc

D_MODEL = 768
MAX_LEN = 8192
NUM_CORES = 2
NUM_SUBCORES = 16
NUM_TILES = NUM_CORES * NUM_SUBCORES
CHUNK = 64
N_SEG = 4


def _pos_encoding(max_len, d_model):
    pos = np.arange(max_len, dtype=np.float32)[:, None]
    i = np.arange(0, d_model, 2, dtype=np.float32)
    div = np.power(10000.0, i / d_model)
    enc = np.zeros((max_len, d_model), dtype=np.float32)
    enc[:, 0::2] = np.sin(pos / div)
    enc[:, 1::2] = np.cos(pos / div)
    return enc


_POS_ENC_NP = _pos_encoding(MAX_LEN, D_MODEL)


def _sc_gather(idx_seg, table, n_rows, d):
    rows_per_tile = n_rows // NUM_TILES
    n_chunks = rows_per_tile // CHUNK

    mesh = plsc.VectorSubcoreMesh(core_axis_name="c", subcore_axis_name="s")

    @functools.partial(
        pl.kernel,
        out_type=jax.ShapeDtypeStruct((n_rows, d), jnp.float32),
        mesh=mesh,
        scratch_types=[
            pltpu.VMEM((rows_per_tile,), jnp.int32),
            pltpu.VMEM((CHUNK, d), jnp.float32),
            pltpu.VMEM((CHUNK, d), jnp.float32),
            pltpu.SemaphoreType.DMA,
            pltpu.SemaphoreType.DMA,
            pltpu.SemaphoreType.DMA,
            pltpu.SemaphoreType.DMA,
            pltpu.SemaphoreType.DMA,
        ],
    )
    def gather_kernel(idx_hbm, table_hbm, out_hbm,
                      idx_v, g0, g1, isem, gsem0, gsem1, wsem0, wsem1):
        wid = lax.axis_index("c") * NUM_SUBCORES + lax.axis_index("s")
        base = wid * rows_per_tile
        idx_cp = pltpu.make_async_copy(
            idx_hbm.at[pl.ds(base, rows_per_tile)], idx_v, isem
        )
        idx_cp.start()
        idx_cp.wait()

        gbufs = (g0, g1)
        gsems = (gsem0, gsem1)
        wsems = (wsem0, wsem1)

        def gather(t, buf, sem):
            return pltpu.make_async_copy(
                table_hbm.at[idx_v.at[pl.ds(t * CHUNK, CHUNK)]], buf, sem
            )

        def writeback(t, buf, sem):
            return pltpu.make_async_copy(
                buf, out_hbm.at[pl.ds(base + t * CHUNK, CHUNK)], sem
            )

        gather(0, g0, gsem0).start()
        gather(1, g1, gsem1).start()
        gather(0, g0, gsem0).wait()
        writeback(0, g0, wsem0).start()

        @pl.loop(0, (n_chunks - 2) // 2)
        def _pair(i):
            for (toff, bsel) in ((1, 1), (2, 0)):
                t = 2 * i + toff
                buf, gsem, wsem = gbufs[bsel], gsems[bsel], wsems[bsel]
                obuf, owsem = gbufs[1 - bsel], wsems[1 - bsel]
                gather(t, buf, gsem).wait()
                writeback(t - 1, obuf, owsem).wait()
                gather(t + 1, obuf, gsems[1 - bsel]).start()
                writeback(t, buf, wsem).start()

        t_last = n_chunks - 1
        gather(t_last, g1, gsem1).wait()
        writeback(t_last - 1, g0, wsem0).wait()
        writeback(t_last, g1, wsem1).start()
        writeback(t_last, g1, wsem1).wait()

    return gather_kernel(idx_seg, table)


def _tc_add_pos(tok_seg, pos_enc, n_rows, seq_len, d):
    block_rows = 512
    pos_blocks = seq_len // block_rows

    def add_kernel(tok_ref, pos_ref, out_ref):
        out_ref[...] = tok_ref[...] + pos_ref[...]

    return pl.pallas_call(
        add_kernel,
        out_shape=jax.ShapeDtypeStruct((n_rows, d), jnp.float32),
        grid=(n_rows // block_rows,),
        in_specs=[
            pl.BlockSpec((block_rows, d), lambda i: (i, 0)),
            pl.BlockSpec((block_rows, d), lambda i: (i % pos_blocks, 0)),
        ],
        out_specs=pl.BlockSpec((block_rows, d), lambda i: (i, 0)),
    )(tok_seg, pos_enc)


def kernel(x, table):
    batch, seq_len = x.shape
    d = table.shape[1]
    pos_enc = jnp.asarray(_POS_ENC_NP[:seq_len])

    idx_flat = x.reshape(batch * seq_len)
    seg_rows = batch * seq_len // N_SEG

    toks = []
    for s in range(N_SEG):
        idx_seg = lax.slice(idx_flat, (s * seg_rows,), ((s + 1) * seg_rows,))
        toks.append(_sc_gather(idx_seg, table, seg_rows, d))
    pos_big = jnp.tile(pos_enc, (N_SEG, 1))
    outs = [_tc_add_pos(lax.slice(pos_big, (s * seg_rows, 0), ((s + 1) * seg_rows, d)), pos_enc, seg_rows, seq_len, d) for s in range(N_SEG)]
    keep = sum(t[0, 0] for t in toks) * 0.0
    out = jnp.concatenate(outs, axis=0) + keep
    return out.reshape(batch, seq_len, d)


# speedup vs baseline: 1.8915x; 1.8915x over previous
import functools

import numpy as np
import jax
import jax.numpy as jnp
from jax import lax
from jax.experimental import pallas as pl
from jax.experimental.pallas import tpu as pltpu
from jax.experimental.pallas import tpu_sc as plsc

D_MODEL = 768
MAX_LEN = 8192
NUM_CORES = 2
NUM_SUBCORES = 16
NUM_TILES = NUM_CORES * NUM_SUBCORES
CHUNK = 64
N_SEG = 1


def _pos_encoding(max_len, d_model):
    pos = np.arange(max_len, dtype=np.float32)[:, None]
    i = np.arange(0, d_model, 2, dtype=np.float32)
    div = np.power(10000.0, i / d_model)
    enc = np.zeros((max_len, d_model), dtype=np.float32)
    enc[:, 0::2] = np.sin(pos / div)
    enc[:, 1::2] = np.cos(pos / div)
    return enc


_POS_ENC_NP = _pos_encoding(MAX_LEN, D_MODEL)


def _sc_gather(idx_seg, table, n_rows, d):
    rows_per_tile = n_rows // NUM_TILES
    n_chunks = rows_per_tile // CHUNK

    mesh = plsc.VectorSubcoreMesh(core_axis_name="c", subcore_axis_name="s")

    @functools.partial(
        pl.kernel,
        out_type=jax.ShapeDtypeStruct((n_rows, d), jnp.float32),
        mesh=mesh,
        scratch_types=[
            pltpu.VMEM((rows_per_tile,), jnp.int32),
            pltpu.VMEM((CHUNK, d), jnp.float32),
            pltpu.VMEM((CHUNK, d), jnp.float32),
            pltpu.SemaphoreType.DMA,
            pltpu.SemaphoreType.DMA,
            pltpu.SemaphoreType.DMA,
            pltpu.SemaphoreType.DMA,
            pltpu.SemaphoreType.DMA,
        ],
    )
    def gather_kernel(idx_hbm, table_hbm, out_hbm,
                      idx_v, g0, g1, isem, gsem0, gsem1, wsem0, wsem1):
        wid = lax.axis_index("c") * NUM_SUBCORES + lax.axis_index("s")
        base = wid * rows_per_tile
        idx_cp = pltpu.make_async_copy(
            idx_hbm.at[pl.ds(base, rows_per_tile)], idx_v, isem
        )
        idx_cp.start()
        idx_cp.wait()

        gbufs = (g0, g1)
        gsems = (gsem0, gsem1)
        wsems = (wsem0, wsem1)

        def gather(t, buf, sem):
            return pltpu.make_async_copy(
                table_hbm.at[idx_v.at[pl.ds(t * CHUNK, CHUNK)]], buf, sem
            )

        def writeback(t, buf, sem):
            return pltpu.make_async_copy(
                buf, out_hbm.at[pl.ds(base + t * CHUNK, CHUNK)], sem
            )

        gather(0, g0, gsem0).start()
        gather(1, g1, gsem1).start()
        gather(0, g0, gsem0).wait()
        writeback(0, g0, wsem0).start()

        @pl.loop(0, (n_chunks - 2) // 2)
        def _pair(i):
            for (toff, bsel) in ((1, 1), (2, 0)):
                t = 2 * i + toff
                buf, gsem, wsem = gbufs[bsel], gsems[bsel], wsems[bsel]
                obuf, owsem = gbufs[1 - bsel], wsems[1 - bsel]
                gather(t, buf, gsem).wait()
                writeback(t - 1, obuf, owsem).wait()
                gather(t + 1, obuf, gsems[1 - bsel]).start()
                writeback(t, buf, wsem).start()

        t_last = n_chunks - 1
        gather(t_last, g1, gsem1).wait()
        writeback(t_last - 1, g0, wsem0).wait()
        writeback(t_last, g1, wsem1).start()
        writeback(t_last, g1, wsem1).wait()

    return gather_kernel(idx_seg, table)


def _tc_add_pos(tok_seg, pos_enc, n_rows, seq_len, d):
    block_rows = 512
    pos_blocks = seq_len // block_rows

    def add_kernel(tok_ref, pos_ref, out_ref):
        out_ref[...] = tok_ref[...] + pos_ref[...]

    return pl.pallas_call(
        add_kernel,
        out_shape=jax.ShapeDtypeStruct((n_rows, d), jnp.float32),
        grid=(n_rows // block_rows,),
        in_specs=[
            pl.BlockSpec((block_rows, d), lambda i: (i, 0)),
            pl.BlockSpec((block_rows, d), lambda i: (i % pos_blocks, 0)),
        ],
        out_specs=pl.BlockSpec((block_rows, d), lambda i: (i, 0)),
    )(tok_seg, pos_enc)


def kernel(x, table):
    batch, seq_len = x.shape
    d = table.shape[1]
    pos_enc = jnp.asarray(_POS_ENC_NP[:seq_len])

    idx_flat = x.reshape(batch * seq_len)
    seg_rows = batch * seq_len // N_SEG

    toks = []
    for s in range(N_SEG):
        idx_seg = lax.slice(idx_flat, (s * seg_rows,), ((s + 1) * seg_rows,))
        toks.append(_sc_gather(idx_seg, table, seg_rows, d))
    outs = [_tc_add_pos(tok, pos_enc, seg_rows, seq_len, d) for tok in toks]
    out = jnp.concatenate(outs, axis=0)
    return out.reshape(batch, seq_len, d)


# speedup vs baseline: 1.9075x; 1.0085x over previous
import functools

import numpy as np
import jax
import jax.numpy as jnp
from jax import lax
from jax.experimental import pallas as pl
from jax.experimental.pallas import tpu as pltpu
from jax.experimental.pallas import tpu_sc as plsc

D_MODEL = 768
MAX_LEN = 8192
NUM_CORES = 2
NUM_SUBCORES = 16
NUM_TILES = NUM_CORES * NUM_SUBCORES
CHUNK = 32
N_SEG = 1


def _pos_encoding(max_len, d_model):
    pos = np.arange(max_len, dtype=np.float32)[:, None]
    i = np.arange(0, d_model, 2, dtype=np.float32)
    div = np.power(10000.0, i / d_model)
    enc = np.zeros((max_len, d_model), dtype=np.float32)
    enc[:, 0::2] = np.sin(pos / div)
    enc[:, 1::2] = np.cos(pos / div)
    return enc


_POS_ENC_NP = _pos_encoding(MAX_LEN, D_MODEL)


N_BUF = 5
AHEAD = 3


def _sc_gather(idx_seg, table, n_rows, d):
    rows_per_tile = n_rows // NUM_TILES
    n_chunks = rows_per_tile // CHUNK

    mesh = plsc.VectorSubcoreMesh(core_axis_name="c", subcore_axis_name="s")

    @functools.partial(
        pl.kernel,
        out_type=jax.ShapeDtypeStruct((n_rows, d), jnp.float32),
        mesh=mesh,
        scratch_types=(
            [pltpu.VMEM((rows_per_tile,), jnp.int32)]
            + [pltpu.VMEM((CHUNK, d), jnp.float32) for _ in range(N_BUF)]
            + [pltpu.SemaphoreType.DMA] * (1 + 2 * N_BUF)
        ),
    )
    def gather_kernel(idx_hbm, table_hbm, out_hbm, idx_v, *rest):
        gbufs = rest[:N_BUF]
        isem = rest[N_BUF]
        gsems = rest[N_BUF + 1:N_BUF + 1 + N_BUF]
        wsems = rest[N_BUF + 1 + N_BUF:]

        wid = lax.axis_index("c") * NUM_SUBCORES + lax.axis_index("s")
        base = wid * rows_per_tile
        idx_cp = pltpu.make_async_copy(
            idx_hbm.at[pl.ds(base, rows_per_tile)], idx_v, isem
        )
        idx_cp.start()
        idx_cp.wait()

        def gather(t):
            b = t % N_BUF
            return pltpu.make_async_copy(
                table_hbm.at[idx_v.at[pl.ds(t * CHUNK, CHUNK)]],
                gbufs[b], gsems[b],
            )

        def writeback(t):
            b = t % N_BUF
            return pltpu.make_async_copy(
                gbufs[b], out_hbm.at[pl.ds(base + t * CHUNK, CHUNK)],
                wsems[b],
            )

        for t in range(AHEAD):
            gather(t).start()
        for t in range(n_chunks):
            gather(t).wait()
            writeback(t).start()
            if t >= 2:
                writeback(t - 2).wait()
            if t + AHEAD < n_chunks:
                gather(t + AHEAD).start()
        writeback(n_chunks - 2).wait()
        writeback(n_chunks - 1).wait()

    return gather_kernel(idx_seg, table)


def _tc_add_pos(tok_seg, pos_enc, n_rows, seq_len, d):
    block_rows = 512
    pos_blocks = seq_len // block_rows

    def add_kernel(tok_ref, pos_ref, out_ref):
        out_ref[...] = tok_ref[...] + pos_ref[...]

    return pl.pallas_call(
        add_kernel,
        out_shape=jax.ShapeDtypeStruct((n_rows, d), jnp.float32),
        grid=(n_rows // block_rows,),
        in_specs=[
            pl.BlockSpec((block_rows, d), lambda i: (i, 0)),
            pl.BlockSpec((block_rows, d), lambda i: (i % pos_blocks, 0)),
        ],
        out_specs=pl.BlockSpec((block_rows, d), lambda i: (i, 0)),
    )(tok_seg, pos_enc)


def kernel(x, table):
    batch, seq_len = x.shape
    d = table.shape[1]
    pos_enc = jnp.asarray(_POS_ENC_NP[:seq_len])

    idx_flat = x.reshape(batch * seq_len)
    seg_rows = batch * seq_len // N_SEG

    toks = []
    for s in range(N_SEG):
        idx_seg = lax.slice(idx_flat, (s * seg_rows,), ((s + 1) * seg_rows,))
        toks.append(_sc_gather(idx_seg, table, seg_rows, d))
    outs = [_tc_add_pos(tok, pos_enc, seg_rows, seq_len, d) for tok in toks]
    out = jnp.concatenate(outs, axis=0)
    return out.reshape(batch, seq_len, d)
